# Initial kernel scaffold; baseline (speedup 1.0000x reference)
import jax
import jax.numpy as jnp
from jax.experimental import pallas as pl


def kernel(length_q, length_k, embeddings_table):
    raise NotImplementedError("write your pallas kernel here")


# speedup vs baseline: 6.4219x; 6.4219x over previous
import jax
import jax.numpy as jnp
from jax import lax
from jax.experimental import pallas as pl
from jax.experimental.pallas import tpu as pltpu
from jax.experimental.pallas import tpu_sc as plsc

D = 64
TROWS = 257
LQ = 2048
LK = 2048
E_ROWS = 4096
CHUNK = 256
HALF = 1024
SEG = HALF + 64
Q_PER_TILE = 64
NLANE = 16
BAND_LO = E_ROWS // 2 - 129


def _sc_body(table_hbm, out_hbm, e_hbm, table_v, seg_v):
    c = lax.axis_index("c")
    s = lax.axis_index("s")

    pltpu.sync_copy(table_hbm, table_v.at[pl.ds(0, TROWS)])
    base_d = s * CHUNK

    def build_row(r, _):
        src = jnp.clip(base_d + r - BAND_LO, 0, TROWS - 1)
        for j in range(D // NLANE):
            seg_v[r, pl.ds(j * NLANE, NLANE)] = table_v[src, pl.ds(j * NLANE, NLANE)]
        return 0

    lax.fori_loop(0, CHUNK, build_row, 0)
    pltpu.sync_copy(seg_v.at[pl.ds(0, CHUNK)], e_hbm.at[c, pl.ds(base_d, CHUNK)])
    plsc.subcore_barrier()

    q0 = c * (LQ // 2) + s * Q_PER_TILE
    for h in range(2):
        d0 = h * HALF + (LQ - Q_PER_TILE) - q0
        pltpu.sync_copy(e_hbm.at[c, pl.ds(d0, SEG)], seg_v)

        def write_row(i, _):
            pltpu.sync_copy(seg_v.at[pl.ds(Q_PER_TILE - 1 - i, HALF)],
                            out_hbm.at[q0 + i, h])
            return 0

        lax.fori_loop(0, Q_PER_TILE, write_row, 0)


def kernel(length_q, length_k, embeddings_table):
    mesh = plsc.VectorSubcoreMesh(core_axis_name="c", subcore_axis_name="s")
    call = pl.kernel(
        _sc_body,
        out_type=(
            jax.ShapeDtypeStruct((LQ, 2, HALF, D), jnp.float32),
            jax.ShapeDtypeStruct((2, E_ROWS, D), jnp.float32),
        ),
        mesh=mesh,
        scratch_types=[
            pltpu.VMEM((TROWS + 7, D), jnp.float32),
            pltpu.VMEM((SEG, D), jnp.float32),
        ],
        compiler_params=pltpu.CompilerParams(use_tc_tiling_on_sc=False),
    )
    out, _ = call(embeddings_table)
    return out.reshape(LQ, LK, D)
